# baseline (speedup 1.0000x reference)
import jax
import jax.numpy as jnp
from jax.experimental import pallas as pl
from jax.experimental.pallas import tpu as pltpu

_ROWS = 32768
_COLS = 1024
_BLK = 512


def _count_body(x_ref, y_ref, nx_ref, ny_ref):
    i = pl.program_id(0)

    @pl.when(i == 0)
    def _init():
        nx_ref[0, 0] = 0
        ny_ref[0, 0] = 0

    nx_ref[0, 0] += jnp.sum((x_ref[...] > 0).astype(jnp.int32))
    ny_ref[0, 0] += jnp.sum((y_ref[...] > 0).astype(jnp.int32))


def kernel(x, y):
    grid = (_ROWS // _BLK,)
    nx, ny = pl.pallas_call(
        _count_body,
        grid=grid,
        in_specs=[
            pl.BlockSpec((_BLK, _COLS), lambda i: (i, 0)),
            pl.BlockSpec((_BLK, _COLS), lambda i: (i, 0)),
        ],
        out_specs=[
            pl.BlockSpec(memory_space=pltpu.SMEM),
            pl.BlockSpec(memory_space=pltpu.SMEM),
        ],
        out_shape=[
            jax.ShapeDtypeStruct((1, 1), jnp.int32),
            jax.ShapeDtypeStruct((1, 1), jnp.int32),
        ],
    )(x, y)
    return jnp.maximum(nx[0, 0], ny[0, 0])
